# baseline (speedup 1.0000x reference)
import functools
import numpy as np
import jax
import jax.numpy as jnp
from jax import lax
from jax.experimental import pallas as pl
from jax.experimental.pallas import tpu as pltpu
from jax.experimental.pallas import tpu_sc as plsc

N_NODES = 10000
N_EDGES = 160000
ATOMIC_DIM = 64
EDGE_DIM = 16
HIDDEN_DIM = 8
RADIUS_CUTOFF = 6.0
DEGREE_NORM = 4.0

_z = np.linspace(-12.0, 12.0, 200001)
_pdf = np.exp(-0.5 * _z ** 2) / np.sqrt(2.0 * np.pi)
_silu = _z / (1.0 + np.exp(-_z))
_SILU_CONST = float(1.0 / np.sqrt(np.sum(_silu ** 2 * _pdf) * (_z[1] - _z[0])))

NC = 2
NS = 16
NW = NC * NS

CPAD = 16
CHUNK = 640
NCH_TOT = N_EDGES // CHUNK
SC_ITERS = (NCH_TOT + NW - 1) // NW
EG = N_EDGES // 8
YB = 400
NB_Y = 8 * (EG // YB)
SCY_ITERS = (NB_Y + NW - 1) // NW
MCHUNK = 640
NCHM = (N_EDGES // 2) // MCHUNK
SCM_ITERS = (2 * NCHM + NW - 1) // NW
NPAD = 10240
RPS = NPAD // NS

MSG_DIM = 32
GDIM = 128


def _sc_gather(coords_hbm, y_hbm, src_hbm, dst_hbm, src8_hbm,
               cs_hbm, cd_hbm, ysp_hbm,
               sidx, didx, qrows, qidx, csv, cdv, ysv, sem):
    wid = lax.axis_index("s") * NC + lax.axis_index("c")

    def body1(i, carry):
        cid = wid + i * NW

        @pl.when(cid < NCH_TOT)
        def _():
            off = cid * CHUNK
            pltpu.sync_copy(src_hbm.at[pl.ds(off, CHUNK)], sidx)
            pltpu.sync_copy(dst_hbm.at[pl.ds(off, CHUNK)], didx)
            hs = []
            for k in range(5):
                sl = pl.ds(128 * k, 128)
                hs.append(pltpu.async_copy(
                    coords_hbm.at[sidx.at[sl]], csv.at[sl], sem))
                hs.append(pltpu.async_copy(
                    coords_hbm.at[didx.at[sl]], cdv.at[sl], sem))
            for h in hs:
                h.wait()
            pltpu.sync_copy(csv, cs_hbm.at[pl.ds(off, CHUNK)])
            pltpu.sync_copy(cdv, cd_hbm.at[pl.ds(off, CHUNK)])

        return carry

    lax.fori_loop(0, SC_ITERS, body1, 0)

    nbs = EG // YB

    def body2(i, carry):
        bid = wid + i * NW

        @pl.when(bid < NB_Y)
        def _():
            p = bid // nbs
            b = bid - p * nbs
            pltpu.sync_copy(src8_hbm.at[pl.ds(b * YB, YB)], qrows)
            iot = lax.iota(jnp.int32, 16)
            cidx = jnp.zeros((16,), jnp.int32) + p

            def colbody(t, carry):
                qidx[pl.ds(t * 16, 16)] = plsc.load_gather(
                    qrows, [t * 16 + iot, cidx])
                return carry

            lax.fori_loop(0, YB // 16, colbody, 0)
            hs = []
            for k in range(5):
                sl = pl.ds(80 * k, 80)
                hs.append(pltpu.async_copy(
                    y_hbm.at[qidx.at[sl]], ysv.at[sl], sem))
            for h in hs:
                h.wait()
            pltpu.sync_copy(ysv, ysp_hbm.at[pl.ds(p * EG + b * YB, YB)])

        return carry

    lax.fori_loop(0, SCY_ITERS, body2, 0)


def _sc_scatter(msga_hbm, msgb_hbm, dst8_hbm, zeros_hbm,
                part_hbm,
                d0, d1, d2, d3, d4, drows, msgv, obuf, acc):
    cidx = lax.axis_index("c")
    sid = lax.axis_index("s")
    wid = sid * NC + cidx
    didx = (d0, d1, d2, d3, d4)

    pltpu.sync_copy(zeros_hbm, acc.at[pl.ds(sid * RPS, RPS)])
    plsc.subcore_barrier()

    iot = lax.iota(jnp.int32, 16)
    riot = lax.shift_right_logical(iot, 2)
    ciot = lax.bitwise_and(iot, 3)

    def do_half(msg_hbm, col0, cid):
        off = cid * MCHUNK
        pltpu.sync_copy(msg_hbm.at[pl.ds(off, MCHUNK)], msgv)
        pltpu.sync_copy(dst8_hbm.at[pl.ds(cid * (MCHUNK // 4), MCHUNK // 4)],
                        drows)
        for k in range(5):
            def colbody(t, carry, k=k):
                didx[k][pl.ds(t * 16, 16)] = plsc.load_gather(
                    drows, [32 * k + t * 4 + riot, ciot + col0])
                return carry

            lax.fori_loop(0, 8, colbody, 0)
        for k in range(5):
            pltpu.sync_copy(msgv.at[pl.ds(128 * k, 128)],
                            acc.at[didx[k]], add=True)

    def body(i, carry):
        cid = wid + i * NW

        @pl.when(cid < NCHM)
        def _():
            do_half(msga_hbm, 0, cid)

        @pl.when((cid >= NCHM) & (cid < 2 * NCHM))
        def _():
            do_half(msgb_hbm, 4, cid - NCHM)

        return carry

    lax.fori_loop(0, SCM_ITERS, body, 0)
    plsc.subcore_barrier()

    pltpu.sync_copy(acc.at[pl.ds(sid * RPS, RPS)], obuf)
    pltpu.sync_copy(obuf, part_hbm.at[cidx, pl.ds(sid * RPS, RPS)])


def _node_body(x_ref, b_ref, y_ref):
    y_ref[...] = jnp.dot(x_ref[...], b_ref[...],
                         preferred_element_type=jnp.float32)


def _edge_body(cs_ref, cd_ref, y0_ref, y1_ref, y2_ref, y3_ref,
               y4_ref, y5_ref, y6_ref, y7_ref,
               w1_ref, q_ref, ra_ref, qb_ref, msga_ref, msgb_ref):
    f32 = jnp.float32
    dn0 = (((0,), (0,)), ((), ()))
    vecp = cs_ref[...] - cd_ref[...]
    r = vecp.shape[0]
    tb = r * 8
    vpt = vecp.T
    vt = jnp.concatenate([vpt[16 * s:16 * s + 16, :] for s in range(8)],
                         axis=1)
    d2t = jnp.sum(vt * vt, axis=0, keepdims=True)
    dt = jnp.sqrt(d2t)
    rint = jnp.where(d2t > 0.0, lax.rsqrt(d2t), 0.0)
    insidet = jnp.where(dt < RADIUS_CUTOFF, rint, 0.0)

    ki = lax.broadcasted_iota(jnp.int32, (EDGE_DIM, tb), 0)
    kft = ki.astype(f32) + 1.0
    argt = kft * (np.pi / RADIUS_CUTOFF) * dt
    ceb = np.sqrt(2.0 / RADIUS_CUTOFF) * np.sqrt(float(EDGE_DIM))
    ebt = ceb * jnp.sin(argt) * insidet

    h0t = lax.dot_general(w1_ref[...], ebt, dn0,
                          preferred_element_type=f32) * 0.25
    sg = 1.0 / (1.0 + jnp.exp(-h0t))
    ht = _SILU_CONST * h0t * sg

    hx = lax.dot_general(ht, q_ref[...], dn0, preferred_element_type=f32)
    unitt = vt * rint
    u = lax.dot_general(unitt, qb_ref[...], dn0, preferred_element_type=f32)
    lane = lax.broadcasted_iota(jnp.int32, (r, MSG_DIM), 1)
    ones = jnp.where(lane < HIDDEN_DIM, 1.0, 0.0)

    ys = (y0_ref, y1_ref, y2_ref, y3_ref, y4_ref, y5_ref, y6_ref, y7_ref)
    pieces = []
    for p in range(8):
        rows = slice(p * r, p * r + r)
        tq_p = jnp.dot(hx[rows, :] * ys[p][...], ra_ref[...],
                       preferred_element_type=f32)
        pieces.append(tq_p * (ones + u[rows, :]))
    msga_ref[...] = jnp.concatenate(pieces[:4], axis=1)
    msgb_ref[...] = jnp.concatenate(pieces[4:], axis=1)


def _combine_body(p_ref, out_ref):
    out_ref[...] = p_ref[0] + p_ref[1]


@functools.lru_cache(maxsize=1)
def _sc_kernels():
    mesh = plsc.VectorSubcoreMesh(
        core_axis_name="c", subcore_axis_name="s",
        num_cores=NC, num_subcores=NS)
    sc_params = pltpu.CompilerParams(use_tc_tiling_on_sc=False,
                                     needs_layout_passes=False)
    gather = pl.kernel(
        _sc_gather,
        compiler_params=sc_params,
        out_type=(
            jax.ShapeDtypeStruct((N_EDGES, CPAD), jnp.float32),
            jax.ShapeDtypeStruct((N_EDGES, CPAD), jnp.float32),
            jax.ShapeDtypeStruct((N_EDGES, GDIM), jnp.float32),
        ),
        mesh=mesh,
        scratch_types=[
            pltpu.VMEM((CHUNK,), jnp.int32),
            pltpu.VMEM((CHUNK,), jnp.int32),
            pltpu.VMEM((YB, 8), jnp.int32),
            pltpu.VMEM((YB,), jnp.int32),
            pltpu.VMEM((CHUNK, CPAD), jnp.float32),
            pltpu.VMEM((CHUNK, CPAD), jnp.float32),
            pltpu.VMEM((YB, GDIM), jnp.float32),
            pltpu.SemaphoreType.DMA,
        ],
    )
    scatter = pl.kernel(
        _sc_scatter,
        compiler_params=sc_params,
        out_type=jax.ShapeDtypeStruct((NC, NPAD, MSG_DIM), jnp.float32),
        mesh=mesh,
        scratch_types=[
            pltpu.VMEM((128,), jnp.int32),
            pltpu.VMEM((128,), jnp.int32),
            pltpu.VMEM((128,), jnp.int32),
            pltpu.VMEM((128,), jnp.int32),
            pltpu.VMEM((128,), jnp.int32),
            pltpu.VMEM((MCHUNK // 4, 8), jnp.int32),
            pltpu.VMEM((MCHUNK, MSG_DIM), jnp.float32),
            pltpu.VMEM((RPS, MSG_DIM), jnp.float32),
            pltpu.VMEM_SHARED((NPAD, MSG_DIM), jnp.float32),
        ],
    )
    return gather, scatter


def kernel(atomic_features, coords, edge_index, W1, W2):
    f32 = jnp.float32
    src = edge_index[0].astype(jnp.int32)
    dst = edge_index[1].astype(jnp.int32)

    coords_p = jnp.zeros((N_NODES, CPAD), f32).at[:, :3].set(coords.astype(f32))

    n0 = ATOMIC_DIM * HIDDEN_DIM
    braw0 = W2[:, :n0].reshape(HIDDEN_DIM, ATOMIC_DIM, HIDDEN_DIM)
    braw0 = braw0.transpose(1, 0, 2).reshape(ATOMIC_DIM, n0 // 8)
    braw1 = W2[:, n0:].reshape(HIDDEN_DIM, ATOMIC_DIM, HIDDEN_DIM)
    braw1 = braw1.transpose(1, 0, 2).reshape(ATOMIC_DIM, n0 // 8)
    s0 = 1.0 / (np.sqrt(4.0 * np.pi) * 8.0 * np.sqrt(8.0) * DEGREE_NORM)
    s1 = np.sqrt(3.0) * s0
    bmat = jnp.concatenate([braw0 * s0, braw1 * s1], axis=1).astype(f32)

    ku = np.arange(HIDDEN_DIM)
    j = np.arange(GDIM)
    o16 = np.arange(2 * HIDDEN_DIM)
    o32 = np.arange(MSG_DIM)
    qm = (ku[:, None] == (j[None, :] % 64) // 8).astype(np.float32)
    rm = ((j[:, None] % 8 == o16[None, :] % 8)
          & ((j[:, None] < 64) == (o16[None, :] < 8))).astype(np.float32)
    qam = (o16[:, None] == np.where(o32[None, :] < 8, o32[None, :],
                                    8 + (o32[None, :] - 8) // 3)
           ).astype(np.float32)
    qbm = np.zeros((CPAD, MSG_DIM), np.float32)
    for c in range(3):
        for v in range(HIDDEN_DIM):
            qbm[c, 8 + 3 * v + c] = 1.0
    ram = rm @ qam

    src8, dst8 = lax.optimization_barrier(
        (src.reshape(N_EDGES // 8, 8), dst.reshape(N_EDGES // 8, 8)))

    y = pl.pallas_call(
        _node_body,
        out_shape=jax.ShapeDtypeStruct((N_NODES, GDIM), f32),
    )(atomic_features.astype(f32), bmat)

    sc_gather, sc_scatter = _sc_kernels()
    cs, cd, ysp = sc_gather(coords_p, y, src, dst, src8)

    cs2 = cs.reshape(N_EDGES // 8, 128)
    cd2 = cd.reshape(N_EDGES // 8, 128)

    TB = 8000
    R = TB // 8
    grid = (N_EDGES // TB,)
    nblk = N_EDGES // TB
    ebs = pl.BlockSpec((R, 128), lambda i: (i, 0))
    ys_specs = [
        pl.BlockSpec((R, 128), functools.partial(
            lambda i, p: (p * nblk + i, 0), p=p))
        for p in range(8)
    ]
    msga8, msgb8 = pl.pallas_call(
        _edge_body,
        grid=grid,
        in_specs=[
            ebs, ebs, *ys_specs,
            pl.BlockSpec((EDGE_DIM, HIDDEN_DIM), lambda i: (0, 0)),
            pl.BlockSpec((HIDDEN_DIM, GDIM), lambda i: (0, 0)),
            pl.BlockSpec((GDIM, MSG_DIM), lambda i: (0, 0)),
            pl.BlockSpec((CPAD, MSG_DIM), lambda i: (0, 0)),
        ],
        out_specs=(ebs, ebs),
        out_shape=(jax.ShapeDtypeStruct((N_EDGES // 8, 128), f32),
                   jax.ShapeDtypeStruct((N_EDGES // 8, 128), f32)),
    )(cs2, cd2, *([ysp] * 8), W1.astype(f32), jnp.asarray(qm),
      jnp.asarray(ram), jnp.asarray(qbm))

    msga = msga8.reshape(N_EDGES // 2, MSG_DIM)
    msgb = msgb8.reshape(N_EDGES // 2, MSG_DIM)

    zeros_rows = jnp.zeros((RPS, MSG_DIM), f32)
    partials = sc_scatter(msga, msgb, dst8, zeros_rows)

    hidden_pad = pl.pallas_call(
        _combine_body,
        out_shape=jax.ShapeDtypeStruct((NPAD // 4, 128), f32),
    )(partials.reshape(NC, NPAD // 4, 128))

    return hidden_pad.reshape(NPAD, MSG_DIM)[:N_NODES]


# speedup vs baseline: 1.0900x; 1.0900x over previous
import functools
import numpy as np
import jax
import jax.numpy as jnp
from jax import lax
from jax.experimental import pallas as pl
from jax.experimental.pallas import tpu as pltpu
from jax.experimental.pallas import tpu_sc as plsc

N_NODES = 10000
N_EDGES = 160000
ATOMIC_DIM = 64
EDGE_DIM = 16
HIDDEN_DIM = 8
RADIUS_CUTOFF = 6.0
DEGREE_NORM = 4.0

_z = np.linspace(-12.0, 12.0, 200001)
_pdf = np.exp(-0.5 * _z ** 2) / np.sqrt(2.0 * np.pi)
_silu = _z / (1.0 + np.exp(-_z))
_SILU_CONST = float(1.0 / np.sqrt(np.sum(_silu ** 2 * _pdf) * (_z[1] - _z[0])))

NC = 2
NS = 16
NW = NC * NS

CPAD = 16
CHUNK = 640
NCH_TOT = N_EDGES // CHUNK
SC_ITERS = (NCH_TOT + NW - 1) // NW
EG = N_EDGES // 8
YB = 400
NB_Y = 8 * (EG // YB)
SCY_ITERS = (NB_Y + NW - 1) // NW
MCHUNK = 640
NCHM = (N_EDGES // 2) // MCHUNK
SCM_ITERS = (2 * NCHM + NW - 1) // NW
NPAD = 10240
RPS = NPAD // NS

MSG_DIM = 32
GDIM = 128


def _sc_gather(coords_hbm, y_hbm, src8_hbm, dst8_hbm,
               cs_hbm, cd_hbm, ysp_hbm,
               srows, drows, sidx, didx, qrows, qidx, csv, cdv, ysv, sem):
    wid = lax.axis_index("s") * NC + lax.axis_index("c")
    iot = lax.iota(jnp.int32, 16)
    riot8 = lax.shift_right_logical(iot, 3)
    ciot8 = lax.bitwise_and(iot, 7)

    def body1(i, carry):
        cid = wid + i * NW

        @pl.when(cid < NCH_TOT)
        def _():
            off = cid * CHUNK
            g0 = cid * (CHUNK // 8)
            pltpu.sync_copy(src8_hbm.at[pl.ds(g0, CHUNK // 8)], srows)
            pltpu.sync_copy(dst8_hbm.at[pl.ds(g0, CHUNK // 8)], drows)

            def unpack(t, carry):
                rr = t * 2 + riot8
                sidx[pl.ds(t * 16, 16)] = plsc.load_gather(
                    srows, [rr, ciot8])
                didx[pl.ds(t * 16, 16)] = plsc.load_gather(
                    drows, [rr, ciot8])
                return carry

            lax.fori_loop(0, CHUNK // 16, unpack, 0)
            hs = []
            for k in range(5):
                sl = pl.ds(128 * k, 128)
                hs.append(pltpu.async_copy(
                    coords_hbm.at[sidx.at[sl]], csv.at[sl], sem))
                hs.append(pltpu.async_copy(
                    coords_hbm.at[didx.at[sl]], cdv.at[sl], sem))
            for h in hs:
                h.wait()
            pltpu.sync_copy(csv, cs_hbm.at[pl.ds(off, CHUNK)])
            pltpu.sync_copy(cdv, cd_hbm.at[pl.ds(off, CHUNK)])

        return carry

    lax.fori_loop(0, SC_ITERS, body1, 0)

    nbs = EG // YB

    def body2(i, carry):
        bid = wid + i * NW

        @pl.when(bid < NB_Y)
        def _():
            p = bid // nbs
            b = bid - p * nbs
            pltpu.sync_copy(src8_hbm.at[pl.ds(b * YB, YB)], qrows)
            cidx = jnp.zeros((16,), jnp.int32) + p

            def colbody(t, carry):
                qidx[pl.ds(t * 16, 16)] = plsc.load_gather(
                    qrows, [t * 16 + iot, cidx])
                return carry

            lax.fori_loop(0, YB // 16, colbody, 0)
            hs = []
            for k in range(5):
                sl = pl.ds(80 * k, 80)
                hs.append(pltpu.async_copy(
                    y_hbm.at[qidx.at[sl]], ysv.at[sl], sem))
            for h in hs:
                h.wait()
            pltpu.sync_copy(ysv, ysp_hbm.at[pl.ds(p * EG + b * YB, YB)])

        return carry

    lax.fori_loop(0, SCY_ITERS, body2, 0)


def _sc_scatter(msga_hbm, msgb_hbm, dst8_hbm, zeros_hbm,
                part_hbm,
                d0, d1, d2, d3, d4, drows, msgv, obuf, acc):
    cidx = lax.axis_index("c")
    sid = lax.axis_index("s")
    wid = sid * NC + cidx
    didx = (d0, d1, d2, d3, d4)

    pltpu.sync_copy(zeros_hbm, acc.at[pl.ds(sid * RPS, RPS)])
    plsc.subcore_barrier()

    iot = lax.iota(jnp.int32, 16)
    riot = lax.shift_right_logical(iot, 2)
    ciot = lax.bitwise_and(iot, 3)

    def do_half(msg_hbm, col0, cid):
        off = cid * MCHUNK
        pltpu.sync_copy(msg_hbm.at[pl.ds(off, MCHUNK)], msgv)
        pltpu.sync_copy(dst8_hbm.at[pl.ds(cid * (MCHUNK // 4), MCHUNK // 4)],
                        drows)
        for k in range(5):
            def colbody(t, carry, k=k):
                didx[k][pl.ds(t * 16, 16)] = plsc.load_gather(
                    drows, [32 * k + t * 4 + riot, ciot + col0])
                return carry

            lax.fori_loop(0, 8, colbody, 0)
        for k in range(5):
            pltpu.sync_copy(msgv.at[pl.ds(128 * k, 128)],
                            acc.at[didx[k]], add=True)

    def body(i, carry):
        cid = wid + i * NW

        @pl.when(cid < NCHM)
        def _():
            do_half(msga_hbm, 0, cid)

        @pl.when((cid >= NCHM) & (cid < 2 * NCHM))
        def _():
            do_half(msgb_hbm, 4, cid - NCHM)

        return carry

    lax.fori_loop(0, SCM_ITERS, body, 0)
    plsc.subcore_barrier()

    pltpu.sync_copy(acc.at[pl.ds(sid * RPS, RPS)], obuf)
    pltpu.sync_copy(obuf, part_hbm.at[cidx, pl.ds(sid * RPS, RPS)])


def _node_body(x_ref, b_ref, y_ref):
    y_ref[...] = jnp.dot(x_ref[...], b_ref[...],
                         preferred_element_type=jnp.float32)


def _edge_body(cs_ref, cd_ref, y0_ref, y1_ref, y2_ref, y3_ref,
               y4_ref, y5_ref, y6_ref, y7_ref,
               w1_ref, q_ref, ra_ref, qb_ref, msga_ref, msgb_ref):
    f32 = jnp.float32
    dn0 = (((0,), (0,)), ((), ()))
    vecp = cs_ref[...] - cd_ref[...]
    r = vecp.shape[0]
    tb = r * 8
    vpt = vecp.T
    vt = jnp.concatenate([vpt[16 * s:16 * s + 16, :] for s in range(8)],
                         axis=1)
    d2t = jnp.sum(vt * vt, axis=0, keepdims=True)
    dt = jnp.sqrt(d2t)
    rint = jnp.where(d2t > 0.0, lax.rsqrt(d2t), 0.0)
    insidet = jnp.where(dt < RADIUS_CUTOFF, rint, 0.0)

    ki = lax.broadcasted_iota(jnp.int32, (EDGE_DIM, tb), 0)
    kft = ki.astype(f32) + 1.0
    argt = kft * (np.pi / RADIUS_CUTOFF) * dt
    ceb = np.sqrt(2.0 / RADIUS_CUTOFF) * np.sqrt(float(EDGE_DIM))
    ebt = ceb * jnp.sin(argt) * insidet

    h0t = lax.dot_general(w1_ref[...], ebt, dn0,
                          preferred_element_type=f32) * 0.25
    sg = 1.0 / (1.0 + jnp.exp(-h0t))
    ht = _SILU_CONST * h0t * sg

    hx = lax.dot_general(ht, q_ref[...], dn0, preferred_element_type=f32)
    unitt = vt * rint
    u = lax.dot_general(unitt, qb_ref[...], dn0, preferred_element_type=f32)
    lane = lax.broadcasted_iota(jnp.int32, (r, MSG_DIM), 1)
    ones = jnp.where(lane < HIDDEN_DIM, 1.0, 0.0)

    ys = (y0_ref, y1_ref, y2_ref, y3_ref, y4_ref, y5_ref, y6_ref, y7_ref)
    pieces = []
    for p in range(8):
        rows = slice(p * r, p * r + r)
        tq_p = jnp.dot(hx[rows, :] * ys[p][...], ra_ref[...],
                       preferred_element_type=f32)
        pieces.append(tq_p * (ones + u[rows, :]))
    msga_ref[...] = jnp.concatenate(pieces[:4], axis=1)
    msgb_ref[...] = jnp.concatenate(pieces[4:], axis=1)


def _combine_body(p_ref, out_ref):
    out_ref[...] = p_ref[0] + p_ref[1]


@functools.lru_cache(maxsize=1)
def _sc_kernels():
    mesh = plsc.VectorSubcoreMesh(
        core_axis_name="c", subcore_axis_name="s",
        num_cores=NC, num_subcores=NS)
    sc_params = pltpu.CompilerParams(use_tc_tiling_on_sc=False,
                                     needs_layout_passes=False)
    gather = pl.kernel(
        _sc_gather,
        compiler_params=sc_params,
        out_type=(
            jax.ShapeDtypeStruct((N_EDGES, CPAD), jnp.float32),
            jax.ShapeDtypeStruct((N_EDGES, CPAD), jnp.float32),
            jax.ShapeDtypeStruct((N_EDGES, GDIM), jnp.float32),
        ),
        mesh=mesh,
        scratch_types=[
            pltpu.VMEM((CHUNK // 8, 8), jnp.int32),
            pltpu.VMEM((CHUNK // 8, 8), jnp.int32),
            pltpu.VMEM((CHUNK,), jnp.int32),
            pltpu.VMEM((CHUNK,), jnp.int32),
            pltpu.VMEM((YB, 8), jnp.int32),
            pltpu.VMEM((YB,), jnp.int32),
            pltpu.VMEM((CHUNK, CPAD), jnp.float32),
            pltpu.VMEM((CHUNK, CPAD), jnp.float32),
            pltpu.VMEM((YB, GDIM), jnp.float32),
            pltpu.SemaphoreType.DMA,
        ],
    )
    scatter = pl.kernel(
        _sc_scatter,
        compiler_params=sc_params,
        out_type=jax.ShapeDtypeStruct((NC, NPAD, MSG_DIM), jnp.float32),
        mesh=mesh,
        scratch_types=[
            pltpu.VMEM((128,), jnp.int32),
            pltpu.VMEM((128,), jnp.int32),
            pltpu.VMEM((128,), jnp.int32),
            pltpu.VMEM((128,), jnp.int32),
            pltpu.VMEM((128,), jnp.int32),
            pltpu.VMEM((MCHUNK // 4, 8), jnp.int32),
            pltpu.VMEM((MCHUNK, MSG_DIM), jnp.float32),
            pltpu.VMEM((RPS, MSG_DIM), jnp.float32),
            pltpu.VMEM_SHARED((NPAD, MSG_DIM), jnp.float32),
        ],
    )
    return gather, scatter


def kernel(atomic_features, coords, edge_index, W1, W2):
    f32 = jnp.float32
    src = edge_index[0].astype(jnp.int32)
    dst = edge_index[1].astype(jnp.int32)

    coords_p = jnp.zeros((N_NODES, CPAD), f32).at[:, :3].set(coords.astype(f32))

    n0 = ATOMIC_DIM * HIDDEN_DIM
    braw0 = W2[:, :n0].reshape(HIDDEN_DIM, ATOMIC_DIM, HIDDEN_DIM)
    braw0 = braw0.transpose(1, 0, 2).reshape(ATOMIC_DIM, n0 // 8)
    braw1 = W2[:, n0:].reshape(HIDDEN_DIM, ATOMIC_DIM, HIDDEN_DIM)
    braw1 = braw1.transpose(1, 0, 2).reshape(ATOMIC_DIM, n0 // 8)
    s0 = 1.0 / (np.sqrt(4.0 * np.pi) * 8.0 * np.sqrt(8.0) * DEGREE_NORM)
    s1 = np.sqrt(3.0) * s0
    bmat = jnp.concatenate([braw0 * s0, braw1 * s1], axis=1).astype(f32)

    ku = np.arange(HIDDEN_DIM)
    j = np.arange(GDIM)
    o16 = np.arange(2 * HIDDEN_DIM)
    o32 = np.arange(MSG_DIM)
    qm = (ku[:, None] == (j[None, :] % 64) // 8).astype(np.float32)
    rm = ((j[:, None] % 8 == o16[None, :] % 8)
          & ((j[:, None] < 64) == (o16[None, :] < 8))).astype(np.float32)
    qam = (o16[:, None] == np.where(o32[None, :] < 8, o32[None, :],
                                    8 + (o32[None, :] - 8) // 3)
           ).astype(np.float32)
    qbm = np.zeros((CPAD, MSG_DIM), np.float32)
    for c in range(3):
        for v in range(HIDDEN_DIM):
            qbm[c, 8 + 3 * v + c] = 1.0
    ram = rm @ qam

    src8 = src.reshape(N_EDGES // 8, 8)
    dst8 = dst.reshape(N_EDGES // 8, 8)

    y = pl.pallas_call(
        _node_body,
        out_shape=jax.ShapeDtypeStruct((N_NODES, GDIM), f32),
    )(atomic_features.astype(f32), bmat)

    sc_gather, sc_scatter = _sc_kernels()
    cs, cd, ysp = sc_gather(coords_p, y, src8, dst8)

    cs2 = cs.reshape(N_EDGES // 8, 128)
    cd2 = cd.reshape(N_EDGES // 8, 128)

    TB = 6400
    R = TB // 8
    grid = (N_EDGES // TB,)
    nblk = N_EDGES // TB
    ebs = pl.BlockSpec((R, 128), lambda i: (i, 0))
    ys_specs = [
        pl.BlockSpec((R, 128), functools.partial(
            lambda i, p: (p * nblk + i, 0), p=p))
        for p in range(8)
    ]
    msga8, msgb8 = pl.pallas_call(
        _edge_body,
        grid=grid,
        in_specs=[
            ebs, ebs, *ys_specs,
            pl.BlockSpec((EDGE_DIM, HIDDEN_DIM), lambda i: (0, 0)),
            pl.BlockSpec((HIDDEN_DIM, GDIM), lambda i: (0, 0)),
            pl.BlockSpec((GDIM, MSG_DIM), lambda i: (0, 0)),
            pl.BlockSpec((CPAD, MSG_DIM), lambda i: (0, 0)),
        ],
        out_specs=(ebs, ebs),
        out_shape=(jax.ShapeDtypeStruct((N_EDGES // 8, 128), f32),
                   jax.ShapeDtypeStruct((N_EDGES // 8, 128), f32)),
    )(cs2, cd2, *([ysp] * 8), W1.astype(f32), jnp.asarray(qm),
      jnp.asarray(ram), jnp.asarray(qbm))

    msga = msga8.reshape(N_EDGES // 2, MSG_DIM)
    msgb = msgb8.reshape(N_EDGES // 2, MSG_DIM)

    zeros_rows = jnp.zeros((RPS, MSG_DIM), f32)
    partials = sc_scatter(msga, msgb, dst8, zeros_rows)

    hidden_pad = pl.pallas_call(
        _combine_body,
        out_shape=jax.ShapeDtypeStruct((NPAD // 4, 128), f32),
    )(partials.reshape(NC, NPAD // 4, 128))

    return hidden_pad.reshape(NPAD, MSG_DIM)[:N_NODES]


# speedup vs baseline: 1.1122x; 1.0204x over previous
import functools
import numpy as np
import jax
import jax.numpy as jnp
from jax import lax
from jax.experimental import pallas as pl
from jax.experimental.pallas import tpu as pltpu
from jax.experimental.pallas import tpu_sc as plsc

N_NODES = 10000
N_EDGES = 160000
ATOMIC_DIM = 64
EDGE_DIM = 16
HIDDEN_DIM = 8
RADIUS_CUTOFF = 6.0
DEGREE_NORM = 4.0

_z = np.linspace(-12.0, 12.0, 200001)
_pdf = np.exp(-0.5 * _z ** 2) / np.sqrt(2.0 * np.pi)
_silu = _z / (1.0 + np.exp(-_z))
_SILU_CONST = float(1.0 / np.sqrt(np.sum(_silu ** 2 * _pdf) * (_z[1] - _z[0])))

NC = 2
NS = 16
NW = NC * NS

CPAD = 16
CHUNK = 1280
NCH_TOT = N_EDGES // CHUNK
SC_ITERS = (NCH_TOT + NW - 1) // NW
EG = N_EDGES // 8
YB = 400
NB_Y = 8 * (EG // YB)
SCY_ITERS = (NB_Y + NW - 1) // NW
MCHUNK = 640
NCHM = (N_EDGES // 2) // MCHUNK
SCM_ITERS = (2 * NCHM + NW - 1) // NW
NPAD = 10240
RPS = NPAD // NS

MSG_DIM = 32
GDIM = 128


def _sc_gather(coords_hbm, y_hbm, src8_hbm, dst8_hbm,
               cs_hbm, cd_hbm, ysp_hbm,
               srows, drows, sidx, didx, qrows, qidx, csv, cdv, ysv, sem):
    wid = lax.axis_index("s") * NC + lax.axis_index("c")
    iot = lax.iota(jnp.int32, 16)
    riot8 = lax.shift_right_logical(iot, 3)
    ciot8 = lax.bitwise_and(iot, 7)

    def body1(i, carry):
        cid = wid + i * NW

        @pl.when(cid < NCH_TOT)
        def _():
            off = cid * CHUNK
            g0 = cid * (CHUNK // 8)
            pltpu.sync_copy(src8_hbm.at[pl.ds(g0, CHUNK // 8)], srows)
            pltpu.sync_copy(dst8_hbm.at[pl.ds(g0, CHUNK // 8)], drows)

            def unpack(t, carry):
                rr = t * 2 + riot8
                sidx[pl.ds(t * 16, 16)] = plsc.load_gather(
                    srows, [rr, ciot8])
                didx[pl.ds(t * 16, 16)] = plsc.load_gather(
                    drows, [rr, ciot8])
                return carry

            lax.fori_loop(0, CHUNK // 16, unpack, 0)
            hs = []
            for k in range(CHUNK // 128):
                sl = pl.ds(128 * k, 128)
                hs.append(pltpu.async_copy(
                    coords_hbm.at[sidx.at[sl]], csv.at[sl], sem))
                hs.append(pltpu.async_copy(
                    coords_hbm.at[didx.at[sl]], cdv.at[sl], sem))
            for h in hs:
                h.wait()
            pltpu.sync_copy(csv, cs_hbm.at[pl.ds(off, CHUNK)])
            pltpu.sync_copy(cdv, cd_hbm.at[pl.ds(off, CHUNK)])

        return carry

    lax.fori_loop(0, SC_ITERS, body1, 0)

    nbs = EG // YB

    def body2(i, carry):
        bid = wid + i * NW

        @pl.when(bid < NB_Y)
        def _():
            p = bid // nbs
            b = bid - p * nbs
            pltpu.sync_copy(src8_hbm.at[pl.ds(b * YB, YB)], qrows)
            cidx = jnp.zeros((16,), jnp.int32) + p

            def colbody(t, carry):
                qidx[pl.ds(t * 16, 16)] = plsc.load_gather(
                    qrows, [t * 16 + iot, cidx])
                return carry

            lax.fori_loop(0, YB // 16, colbody, 0)
            hs = []
            for k in range(5):
                sl = pl.ds(80 * k, 80)
                hs.append(pltpu.async_copy(
                    y_hbm.at[qidx.at[sl]], ysv.at[sl], sem))
            for h in hs:
                h.wait()
            pltpu.sync_copy(ysv, ysp_hbm.at[pl.ds(p * EG + b * YB, YB)])

        return carry

    lax.fori_loop(0, SCY_ITERS, body2, 0)


def _sc_scatter(msga_hbm, msgb_hbm, dst8_hbm, zeros_hbm,
                part_hbm,
                d0, d1, d2, d3, d4, drows, msgv, obuf, acc):
    cidx = lax.axis_index("c")
    sid = lax.axis_index("s")
    wid = sid * NC + cidx
    didx = (d0, d1, d2, d3, d4)

    pltpu.sync_copy(zeros_hbm, acc.at[pl.ds(sid * RPS, RPS)])
    plsc.subcore_barrier()

    iot = lax.iota(jnp.int32, 16)
    riot = lax.shift_right_logical(iot, 2)
    ciot = lax.bitwise_and(iot, 3)

    def do_half(msg_hbm, col0, cid):
        off = cid * MCHUNK
        pltpu.sync_copy(msg_hbm.at[pl.ds(off, MCHUNK)], msgv)
        pltpu.sync_copy(dst8_hbm.at[pl.ds(cid * (MCHUNK // 4), MCHUNK // 4)],
                        drows)
        for k in range(5):
            def colbody(t, carry, k=k):
                didx[k][pl.ds(t * 16, 16)] = plsc.load_gather(
                    drows, [32 * k + t * 4 + riot, ciot + col0])
                return carry

            lax.fori_loop(0, 8, colbody, 0)
        for k in range(5):
            pltpu.sync_copy(msgv.at[pl.ds(128 * k, 128)],
                            acc.at[didx[k]], add=True)

    def body(i, carry):
        cid = wid + i * NW

        @pl.when(cid < NCHM)
        def _():
            do_half(msga_hbm, 0, cid)

        @pl.when((cid >= NCHM) & (cid < 2 * NCHM))
        def _():
            do_half(msgb_hbm, 4, cid - NCHM)

        return carry

    lax.fori_loop(0, SCM_ITERS, body, 0)
    plsc.subcore_barrier()

    pltpu.sync_copy(acc.at[pl.ds(sid * RPS, RPS)], obuf)
    pltpu.sync_copy(obuf, part_hbm.at[cidx, pl.ds(sid * RPS, RPS)])


def _node_body(x_ref, b_ref, y_ref):
    y_ref[...] = jnp.dot(x_ref[...], b_ref[...],
                         preferred_element_type=jnp.float32)


def _edge_body(cs_ref, cd_ref, y0_ref, y1_ref, y2_ref, y3_ref,
               y4_ref, y5_ref, y6_ref, y7_ref,
               w1_ref, q_ref, ra_ref, qb_ref, msga_ref, msgb_ref):
    f32 = jnp.float32
    dn0 = (((0,), (0,)), ((), ()))
    vecp = cs_ref[...] - cd_ref[...]
    r = vecp.shape[0]
    tb = r * 8
    vpt = vecp.T
    vt = jnp.concatenate([vpt[16 * s:16 * s + 16, :] for s in range(8)],
                         axis=1)
    d2t = jnp.sum(vt * vt, axis=0, keepdims=True)
    dt = jnp.sqrt(d2t)
    rint = jnp.where(d2t > 0.0, lax.rsqrt(d2t), 0.0)
    insidet = jnp.where(dt < RADIUS_CUTOFF, rint, 0.0)

    ki = lax.broadcasted_iota(jnp.int32, (EDGE_DIM, tb), 0)
    kft = ki.astype(f32) + 1.0
    argt = kft * (np.pi / RADIUS_CUTOFF) * dt
    ceb = np.sqrt(2.0 / RADIUS_CUTOFF) * np.sqrt(float(EDGE_DIM))
    ebt = ceb * jnp.sin(argt) * insidet

    h0t = lax.dot_general(w1_ref[...], ebt, dn0,
                          preferred_element_type=f32) * 0.25
    sg = 1.0 / (1.0 + jnp.exp(-h0t))
    ht = _SILU_CONST * h0t * sg

    hx = lax.dot_general(ht, q_ref[...], dn0, preferred_element_type=f32)
    unitt = vt * rint
    u = lax.dot_general(unitt, qb_ref[...], dn0, preferred_element_type=f32)
    lane = lax.broadcasted_iota(jnp.int32, (r, MSG_DIM), 1)
    ones = jnp.where(lane < HIDDEN_DIM, 1.0, 0.0)

    ys = (y0_ref, y1_ref, y2_ref, y3_ref, y4_ref, y5_ref, y6_ref, y7_ref)
    pieces = []
    for p in range(8):
        rows = slice(p * r, p * r + r)
        tq_p = jnp.dot(hx[rows, :] * ys[p][...], ra_ref[...],
                       preferred_element_type=f32)
        pieces.append(tq_p * (ones + u[rows, :]))
    msga_ref[...] = jnp.concatenate(pieces[:4], axis=1)
    msgb_ref[...] = jnp.concatenate(pieces[4:], axis=1)


def _combine_body(p_ref, out_ref):
    out_ref[...] = p_ref[0] + p_ref[1]


@functools.lru_cache(maxsize=1)
def _sc_kernels():
    mesh = plsc.VectorSubcoreMesh(
        core_axis_name="c", subcore_axis_name="s",
        num_cores=NC, num_subcores=NS)
    sc_params = pltpu.CompilerParams(use_tc_tiling_on_sc=False,
                                     needs_layout_passes=False)
    gather = pl.kernel(
        _sc_gather,
        compiler_params=sc_params,
        out_type=(
            jax.ShapeDtypeStruct((N_EDGES, CPAD), jnp.float32),
            jax.ShapeDtypeStruct((N_EDGES, CPAD), jnp.float32),
            jax.ShapeDtypeStruct((N_EDGES, GDIM), jnp.float32),
        ),
        mesh=mesh,
        scratch_types=[
            pltpu.VMEM((CHUNK // 8, 8), jnp.int32),
            pltpu.VMEM((CHUNK // 8, 8), jnp.int32),
            pltpu.VMEM((CHUNK,), jnp.int32),
            pltpu.VMEM((CHUNK,), jnp.int32),
            pltpu.VMEM((YB, 8), jnp.int32),
            pltpu.VMEM((YB,), jnp.int32),
            pltpu.VMEM((CHUNK, CPAD), jnp.float32),
            pltpu.VMEM((CHUNK, CPAD), jnp.float32),
            pltpu.VMEM((YB, GDIM), jnp.float32),
            pltpu.SemaphoreType.DMA,
        ],
    )
    scatter = pl.kernel(
        _sc_scatter,
        compiler_params=sc_params,
        out_type=jax.ShapeDtypeStruct((NC, NPAD, MSG_DIM), jnp.float32),
        mesh=mesh,
        scratch_types=[
            pltpu.VMEM((128,), jnp.int32),
            pltpu.VMEM((128,), jnp.int32),
            pltpu.VMEM((128,), jnp.int32),
            pltpu.VMEM((128,), jnp.int32),
            pltpu.VMEM((128,), jnp.int32),
            pltpu.VMEM((MCHUNK // 4, 8), jnp.int32),
            pltpu.VMEM((MCHUNK, MSG_DIM), jnp.float32),
            pltpu.VMEM((RPS, MSG_DIM), jnp.float32),
            pltpu.VMEM_SHARED((NPAD, MSG_DIM), jnp.float32),
        ],
    )
    return gather, scatter


def kernel(atomic_features, coords, edge_index, W1, W2):
    f32 = jnp.float32
    src = edge_index[0].astype(jnp.int32)
    dst = edge_index[1].astype(jnp.int32)

    coords_p = jnp.zeros((N_NODES, CPAD), f32).at[:, :3].set(coords.astype(f32))

    n0 = ATOMIC_DIM * HIDDEN_DIM
    braw0 = W2[:, :n0].reshape(HIDDEN_DIM, ATOMIC_DIM, HIDDEN_DIM)
    braw0 = braw0.transpose(1, 0, 2).reshape(ATOMIC_DIM, n0 // 8)
    braw1 = W2[:, n0:].reshape(HIDDEN_DIM, ATOMIC_DIM, HIDDEN_DIM)
    braw1 = braw1.transpose(1, 0, 2).reshape(ATOMIC_DIM, n0 // 8)
    s0 = 1.0 / (np.sqrt(4.0 * np.pi) * 8.0 * np.sqrt(8.0) * DEGREE_NORM)
    s1 = np.sqrt(3.0) * s0
    bmat = jnp.concatenate([braw0 * s0, braw1 * s1], axis=1).astype(f32)

    ku = np.arange(HIDDEN_DIM)
    j = np.arange(GDIM)
    o16 = np.arange(2 * HIDDEN_DIM)
    o32 = np.arange(MSG_DIM)
    qm = (ku[:, None] == (j[None, :] % 64) // 8).astype(np.float32)
    rm = ((j[:, None] % 8 == o16[None, :] % 8)
          & ((j[:, None] < 64) == (o16[None, :] < 8))).astype(np.float32)
    qam = (o16[:, None] == np.where(o32[None, :] < 8, o32[None, :],
                                    8 + (o32[None, :] - 8) // 3)
           ).astype(np.float32)
    qbm = np.zeros((CPAD, MSG_DIM), np.float32)
    for c in range(3):
        for v in range(HIDDEN_DIM):
            qbm[c, 8 + 3 * v + c] = 1.0
    ram = rm @ qam

    src8 = src.reshape(N_EDGES // 8, 8)
    dst8 = dst.reshape(N_EDGES // 8, 8)

    y = pl.pallas_call(
        _node_body,
        out_shape=jax.ShapeDtypeStruct((N_NODES, GDIM), f32),
    )(atomic_features.astype(f32), bmat)

    sc_gather, sc_scatter = _sc_kernels()
    cs, cd, ysp = sc_gather(coords_p, y, src8, dst8)

    cs2 = cs.reshape(N_EDGES // 8, 128)
    cd2 = cd.reshape(N_EDGES // 8, 128)

    TB = 6400
    R = TB // 8
    grid = (N_EDGES // TB,)
    nblk = N_EDGES // TB
    ebs = pl.BlockSpec((R, 128), lambda i: (i, 0))
    ys_specs = [
        pl.BlockSpec((R, 128), functools.partial(
            lambda i, p: (p * nblk + i, 0), p=p))
        for p in range(8)
    ]
    msga8, msgb8 = pl.pallas_call(
        _edge_body,
        grid=grid,
        in_specs=[
            ebs, ebs, *ys_specs,
            pl.BlockSpec((EDGE_DIM, HIDDEN_DIM), lambda i: (0, 0)),
            pl.BlockSpec((HIDDEN_DIM, GDIM), lambda i: (0, 0)),
            pl.BlockSpec((GDIM, MSG_DIM), lambda i: (0, 0)),
            pl.BlockSpec((CPAD, MSG_DIM), lambda i: (0, 0)),
        ],
        out_specs=(ebs, ebs),
        out_shape=(jax.ShapeDtypeStruct((N_EDGES // 8, 128), f32),
                   jax.ShapeDtypeStruct((N_EDGES // 8, 128), f32)),
    )(cs2, cd2, *([ysp] * 8), W1.astype(f32), jnp.asarray(qm),
      jnp.asarray(ram), jnp.asarray(qbm))

    msga = msga8.reshape(N_EDGES // 2, MSG_DIM)
    msgb = msgb8.reshape(N_EDGES // 2, MSG_DIM)

    zeros_rows = jnp.zeros((RPS, MSG_DIM), f32)
    partials = sc_scatter(msga, msgb, dst8, zeros_rows)

    hidden_pad = pl.pallas_call(
        _combine_body,
        out_shape=jax.ShapeDtypeStruct((NPAD // 4, 128), f32),
    )(partials.reshape(NC, NPAD // 4, 128))

    return hidden_pad.reshape(NPAD, MSG_DIM)[:N_NODES]
